# baseline (speedup 1.0000x reference)
import functools

import jax
import jax.numpy as jnp
from jax import lax
from jax.experimental import pallas as pl
from jax.experimental.pallas import tpu as pltpu
from jax.experimental.pallas import tpu_sc as plsc

DATA_DIM = 28
ROWW = 32
N_NODES = 524288
N_CORNERS = 600000
N_QUERIES = 262144

NC = 2
NS = 16
L = 16
NW = NC * NS
QPW = N_QUERIES // NW
CHUNK = 128
NCHUNK = QPW // CHUNK

RPT = N_CORNERS // NS
PR = 750
NPAD = RPT // PR


def _body(indices_hbm, nids_hbm, data_hbm, weights_hbm, out_hbm, padt_hbm,
          idx_v, cid_v, cflat_v, rows_v, w_v, out_v, pin_v, pout_v,
          sem_n, sem_d):
    sid = lax.axis_index("s")
    wid = sid * NC + lax.axis_index("c")
    base = wid * QPW

    iota = lax.iota(jnp.int32, L)
    hi = iota >> 3
    lo = iota & 7

    def pad_body(i, _):
        rbase = pl.multiple_of(sid * RPT + i * PR, 2)
        pltpu.sync_copy(data_hbm.at[pl.ds(rbase, PR), :], pin_v)

        def widen(r, _):
            pout_v[r, pl.ds(0, L)] = pin_v[r, pl.ds(0, L)]
            pout_v[r, pl.ds(DATA_DIM - L, L)] = pin_v[r, pl.ds(DATA_DIM - L, L)]
            return 0

        lax.fori_loop(0, PR, widen, 0, unroll=8)
        pltpu.sync_copy(pout_v, padt_hbm.at[pl.ds(rbase, PR), :])
        return 0

    lax.fori_loop(0, NPAD, pad_body, 0)
    plsc.subcore_barrier()

    def chunk_body(g, _):
        qbase = pl.multiple_of(base + g * CHUNK, CHUNK)
        pltpu.sync_copy(indices_hbm.at[pl.ds(qbase, CHUNK)], idx_v)
        pltpu.async_copy(nids_hbm.at[idx_v], cid_v, sem_n).wait()
        for t in range(CHUNK // 2):
            idx_c = 2 * t + hi
            cvec = plsc.load_gather(cid_v, [idx_c, lo])
            cflat_v[t // 8, pl.ds((t % 8) * L, L)] = cvec
        copies = [
            pltpu.async_copy(padt_hbm.at[cflat_v.at[k]], rows_v.at[k], sem_d)
            for k in range(8)
        ]
        for c in copies:
            c.wait()
        pltpu.sync_copy(weights_hbm.at[pl.ds(qbase, CHUNK), :], w_v)

        def q_body(c2, _):
            k = c2 // 8
            m = (c2 % 8) * L
            wv = plsc.load_gather(w_v, [2 * c2 + hi, lo])
            for h, c in ((0, 2 * c2), (8, 2 * c2 + 1)):
                w0 = wv[h]
                acc_lo = w0 * rows_v[k, m + h, pl.ds(0, L)]
                acc_hi = w0 * rows_v[k, m + h, pl.ds(DATA_DIM - L, L)]
                for j in range(1, 8):
                    wj = wv[h + j]
                    acc_lo = acc_lo + wj * rows_v[k, m + h + j, pl.ds(0, L)]
                    acc_hi = acc_hi + wj * rows_v[k, m + h + j, pl.ds(DATA_DIM - L, L)]
                out_v[c, pl.ds(0, L)] = acc_lo
                out_v[c, pl.ds(DATA_DIM - L, L)] = acc_hi
            return 0

        lax.fori_loop(0, CHUNK // 2, q_body, 0)
        pltpu.sync_copy(out_v, out_hbm.at[pl.ds(qbase, CHUNK), :])
        return 0

    lax.fori_loop(0, NCHUNK, chunk_body, 0)


@jax.jit
def kernel(indices, nids, data, weights):
    mesh = plsc.VectorSubcoreMesh(core_axis_name="c", subcore_axis_name="s")
    run = functools.partial(
        pl.kernel,
        mesh=mesh,
        out_type=(
            jax.ShapeDtypeStruct((N_QUERIES, DATA_DIM), jnp.float32),
            jax.ShapeDtypeStruct((N_CORNERS, ROWW), jnp.float32),
        ),
        compiler_params=pltpu.CompilerParams(
            needs_layout_passes=False, use_tc_tiling_on_sc=False),
        scratch_types=[
            pltpu.VMEM((CHUNK,), jnp.int32),
            pltpu.VMEM((CHUNK, 8), jnp.int32),
            pltpu.VMEM((8, CHUNK), jnp.int32),
            pltpu.VMEM((8, CHUNK, ROWW), jnp.float32),
            pltpu.VMEM((CHUNK, 8), jnp.float32),
            pltpu.VMEM((CHUNK, DATA_DIM), jnp.float32),
            pltpu.VMEM((PR, DATA_DIM), jnp.float32),
            pltpu.VMEM((PR, ROWW), jnp.float32),
            pltpu.SemaphoreType.DMA,
            pltpu.SemaphoreType.DMA,
        ],
    )(_body)
    out, _ = run(indices, nids, data, weights)
    return out


# speedup vs baseline: 1.0907x; 1.0907x over previous
import functools

import jax
import jax.numpy as jnp
from jax import lax
from jax.experimental import pallas as pl
from jax.experimental.pallas import tpu as pltpu
from jax.experimental.pallas import tpu_sc as plsc

DATA_DIM = 28
ROWW = 32
N_NODES = 524288
N_CORNERS = 600000
N_QUERIES = 262144

NC = 2
NS = 16
L = 16
NW = NC * NS
QPW = N_QUERIES // NW
CHUNK = 128
NCHUNK = QPW // CHUNK

RPT = N_CORNERS // NS
PR = 250
NPAD = RPT // PR


def _body(indices_hbm, nids_hbm, data_hbm, weights_hbm, out_hbm, padt_hbm,
          idx_v, cid_v, cflat_v, rows_v, w_v, out_v, pin_v, pout_v,
          sem_n, sem_d, sem_pi, sem_po):
    sid = lax.axis_index("s")
    wid = sid * NC + lax.axis_index("c")
    base = wid * QPW

    iota = lax.iota(jnp.int32, L)
    hi = iota >> 3
    lo = iota & 7

    def pad_in(i, b):
        rbase = pl.multiple_of(sid * RPT + i * PR, 2)
        return pltpu.make_async_copy(
            data_hbm.at[pl.ds(rbase, PR), :], pin_v.at[b], sem_pi)

    def pad_out(i, b):
        rbase = pl.multiple_of(sid * RPT + i * PR, 2)
        return pltpu.make_async_copy(
            pout_v.at[b], padt_hbm.at[pl.ds(rbase, PR), :], sem_po)

    def widen(b):
        def w(r, _):
            pout_v[b, r, pl.ds(0, L)] = pin_v[b, r, pl.ds(0, L)]
            pout_v[b, r, pl.ds(DATA_DIM - L, L)] = \
                pin_v[b, r, pl.ds(DATA_DIM - L, L)]
            return 0

        lax.fori_loop(0, PR, w, 0, unroll=8)

    pad_in(0, 0).start()
    pad_in(1, 1).start()
    for i in (0, 1):
        pad_in(i, i).wait()
        widen(i)
        pad_out(i, i).start()
        pad_in(i + 2, i).start()

    def pad_body(ii, _):
        for b in range(2):
            i = 2 * ii + b
            pad_in(i, b).wait()
            pad_out(i - 2, b).wait()
            widen(b)
            pad_out(i, b).start()
            pad_in(i + 2, b).start()
        return 0

    lax.fori_loop(1, NPAD // 2 - 1, pad_body, 0)
    for b in range(2):
        i = NPAD - 2 + b
        pad_in(i, b).wait()
        pad_out(i - 2, b).wait()
        widen(b)
        pad_out(i, b).start()
    pad_out(NPAD - 2, 0).wait()
    pad_out(NPAD - 1, 1).wait()
    plsc.subcore_barrier()

    def chunk_body(g, _):
        qbase = pl.multiple_of(base + g * CHUNK, CHUNK)
        pltpu.sync_copy(indices_hbm.at[pl.ds(qbase, CHUNK)], idx_v)
        pltpu.async_copy(nids_hbm.at[idx_v], cid_v, sem_n).wait()
        for t in range(CHUNK // 2):
            idx_c = 2 * t + hi
            cvec = plsc.load_gather(cid_v, [idx_c, lo])
            cflat_v[t // 8, pl.ds((t % 8) * L, L)] = cvec
        copies = [
            pltpu.async_copy(padt_hbm.at[cflat_v.at[k]], rows_v.at[k], sem_d)
            for k in range(8)
        ]
        for c in copies:
            c.wait()
        pltpu.sync_copy(weights_hbm.at[pl.ds(qbase, CHUNK), :], w_v)

        def q_body(c2, _):
            k = c2 // 8
            m = (c2 % 8) * L
            wv = plsc.load_gather(w_v, [2 * c2 + hi, lo])
            for h, c in ((0, 2 * c2), (8, 2 * c2 + 1)):
                w0 = wv[h]
                acc_lo = w0 * rows_v[k, m + h, pl.ds(0, L)]
                acc_hi = w0 * rows_v[k, m + h, pl.ds(DATA_DIM - L, L)]
                for j in range(1, 8):
                    wj = wv[h + j]
                    acc_lo = acc_lo + wj * rows_v[k, m + h + j, pl.ds(0, L)]
                    acc_hi = acc_hi + wj * rows_v[k, m + h + j, pl.ds(DATA_DIM - L, L)]
                out_v[c, pl.ds(0, L)] = acc_lo
                out_v[c, pl.ds(DATA_DIM - L, L)] = acc_hi
            return 0

        lax.fori_loop(0, CHUNK // 2, q_body, 0)
        pltpu.sync_copy(out_v, out_hbm.at[pl.ds(qbase, CHUNK), :])
        return 0

    lax.fori_loop(0, NCHUNK, chunk_body, 0)


@jax.jit
def kernel(indices, nids, data, weights):
    mesh = plsc.VectorSubcoreMesh(core_axis_name="c", subcore_axis_name="s")
    run = functools.partial(
        pl.kernel,
        mesh=mesh,
        out_type=(
            jax.ShapeDtypeStruct((N_QUERIES, DATA_DIM), jnp.float32),
            jax.ShapeDtypeStruct((N_CORNERS, ROWW), jnp.float32),
        ),
        compiler_params=pltpu.CompilerParams(
            needs_layout_passes=False, use_tc_tiling_on_sc=False),
        scratch_types=[
            pltpu.VMEM((CHUNK,), jnp.int32),
            pltpu.VMEM((CHUNK, 8), jnp.int32),
            pltpu.VMEM((8, CHUNK), jnp.int32),
            pltpu.VMEM((8, CHUNK, ROWW), jnp.float32),
            pltpu.VMEM((CHUNK, 8), jnp.float32),
            pltpu.VMEM((CHUNK, DATA_DIM), jnp.float32),
            pltpu.VMEM((2, PR, DATA_DIM), jnp.float32),
            pltpu.VMEM((2, PR, ROWW), jnp.float32),
            pltpu.SemaphoreType.DMA,
            pltpu.SemaphoreType.DMA,
            pltpu.SemaphoreType.DMA,
            pltpu.SemaphoreType.DMA,
        ],
    )(_body)
    out, _ = run(indices, nids, data, weights)
    return out


# speedup vs baseline: 1.6187x; 1.4841x over previous
import functools

import jax
import jax.numpy as jnp
from jax import lax
from jax.experimental import pallas as pl
from jax.experimental.pallas import tpu as pltpu
from jax.experimental.pallas import tpu_sc as plsc

DATA_DIM = 28
ROWW = 32
N_NODES = 524288
N_CORNERS = 600000
N_QUERIES = 262144

NC = 2
NS = 16
L = 16
NW = NC * NS
QPW = N_QUERIES // NW
CHUNK = 128
NCHUNK = QPW // CHUNK


def _body(indices_hbm, nids_hbm, data_hbm, weights_hbm, out_hbm,
          idx_v, cid_v, cflat_v, rows_v, w_v, out_v,
          sem_i, sem_n, sem_d, sem_w):
    wid = lax.axis_index("s") * NC + lax.axis_index("c")
    base = wid * QPW

    iota = lax.iota(jnp.int32, L)
    hi = iota >> 3
    lo = iota & 7

    def qb(g):
        gc = lax.min(g, NCHUNK - 1)
        return pl.multiple_of(base + gc * CHUNK, CHUNK)

    def idx_copy(g, p):
        return pltpu.make_async_copy(
            indices_hbm.at[pl.ds(qb(g), CHUNK)], idx_v.at[p], sem_i)

    def nids_copy(g, p):
        return pltpu.make_async_copy(nids_hbm.at[idx_v.at[p]],
                                     cid_v.at[p], sem_n)

    def w_copy(g, p):
        return pltpu.make_async_copy(
            weights_hbm.at[pl.ds(qb(g), CHUNK), :], w_v.at[p], sem_w)

    def repack(p):
        for t in range(CHUNK // 2):
            idx_c = 2 * t + hi
            cvec = plsc.load_gather(cid_v.at[p], [idx_c, lo])
            cflat_v[p, t // 8, pl.ds((t % 8) * L, L)] = cvec

    def fire_rows(p):
        for k in range(8):
            pltpu.async_copy(data_hbm.at[cflat_v.at[p, k]],
                             rows_v.at[p, k], sem_d)

    def wait_rows(p):
        for k in range(8):
            pltpu.make_async_copy(data_hbm.at[cflat_v.at[p, k]],
                                  rows_v.at[p, k], sem_d).wait()

    def compute(g, p):
        def q_body(c2, _):
            k = c2 // 8
            m = (c2 % 8) * L
            wv = plsc.load_gather(w_v.at[p], [2 * c2 + hi, lo])
            for h, c in ((0, 2 * c2), (8, 2 * c2 + 1)):
                w0 = wv[h]
                acc_lo = w0 * rows_v[p, k, m + h, pl.ds(0, L)]
                acc_hi = w0 * rows_v[p, k, m + h, pl.ds(DATA_DIM - L, L)]
                for j in range(1, 8):
                    wj = wv[h + j]
                    acc_lo = acc_lo + wj * rows_v[p, k, m + h + j, pl.ds(0, L)]
                    acc_hi = acc_hi + wj * rows_v[p, k, m + h + j,
                                                  pl.ds(DATA_DIM - L, L)]
                out_v[c, pl.ds(0, L)] = acc_lo
                out_v[c, pl.ds(DATA_DIM - L, L)] = acc_hi
            return 0

        lax.fori_loop(0, CHUNK // 2, q_body, 0)
        pltpu.sync_copy(out_v, out_hbm.at[pl.ds(qb(g), CHUNK), :])

    pltpu.sync_copy(indices_hbm.at[pl.ds(qb(0), CHUNK)], idx_v.at[0])
    nids_copy(0, 0).start()
    idx_copy(1, 1).start()
    w_copy(0, 0).start()
    nids_copy(0, 0).wait()
    repack(0)
    fire_rows(0)
    idx_copy(1, 1).wait()
    nids_copy(1, 1).start()
    idx_copy(2, 0).start()
    w_copy(1, 1).start()

    def body(s, _):
        p = s & 1
        nids_copy(s, p).wait()
        repack(p)
        wait_rows(1 - p)
        fire_rows(p)
        idx_copy(s + 1, 1 - p).wait()
        nids_copy(s + 1, 1 - p).start()
        idx_copy(s + 2, p).start()
        w_copy(s - 1, 1 - p).wait()
        compute(s - 1, 1 - p)
        w_copy(s + 1, 1 - p).start()
        return 0

    lax.fori_loop(1, NCHUNK, body, 0)

    wait_rows(1)
    w_copy(NCHUNK - 1, 1).wait()
    compute(NCHUNK - 1, 1)
    nids_copy(NCHUNK, 0).wait()
    idx_copy(NCHUNK + 1, 1).wait()
    w_copy(NCHUNK, 0).wait()


@jax.jit
def kernel(indices, nids, data, weights):
    mesh = plsc.VectorSubcoreMesh(core_axis_name="c", subcore_axis_name="s")
    run = functools.partial(
        pl.kernel,
        mesh=mesh,
        out_type=jax.ShapeDtypeStruct((N_QUERIES, DATA_DIM), jnp.float32),
        compiler_params=pltpu.CompilerParams(
            needs_layout_passes=False, use_tc_tiling_on_sc=False),
        scratch_types=[
            pltpu.VMEM((2, CHUNK), jnp.int32),
            pltpu.VMEM((2, CHUNK, 8), jnp.int32),
            pltpu.VMEM((2, 8, CHUNK), jnp.int32),
            pltpu.VMEM((2, 8, CHUNK, ROWW), jnp.float32),
            pltpu.VMEM((2, CHUNK, 8), jnp.float32),
            pltpu.VMEM((CHUNK, DATA_DIM), jnp.float32),
            pltpu.SemaphoreType.DMA,
            pltpu.SemaphoreType.DMA,
            pltpu.SemaphoreType.DMA,
            pltpu.SemaphoreType.DMA,
        ],
    )(_body)
    data_p = jnp.concatenate(
        [data, jnp.zeros((N_CORNERS, ROWW - DATA_DIM), jnp.float32)], axis=1)
    return run(indices, nids, data_p, weights)
